# baseline (speedup 1.0000x reference)
import jax
from jax.experimental import pallas as pl
from jax.experimental.pallas import tpu as pltpu

_BLOCK_ROWS = 256


def _copy_body(pe_ref, out_ref):
    out_ref[...] = pe_ref[...]


def kernel(x, pe):
    seq_len = x.shape[1]
    d_model = pe.shape[2]
    grid = (seq_len // _BLOCK_ROWS,)
    out_shape = jax.ShapeDtypeStruct((1, seq_len, d_model), pe.dtype)
    return pl.pallas_call(
        _copy_body,
        grid=grid,
        in_specs=[pl.BlockSpec((1, _BLOCK_ROWS, d_model), lambda i: (0, i, 0))],
        out_specs=pl.BlockSpec((1, _BLOCK_ROWS, d_model), lambda i: (0, i, 0)),
        out_shape=out_shape,
    )(pe)


# speedup vs baseline: 1.5414x; 1.5414x over previous
import jax
from jax.experimental import pallas as pl
from jax.experimental.pallas import tpu as pltpu

_BLOCK_ROWS = 1024


def _copy_body(pe_ref, out_ref):
    out_ref[...] = pe_ref[...]


def kernel(x, pe):
    seq_len = x.shape[1]
    d_model = pe.shape[2]
    grid = (seq_len // _BLOCK_ROWS,)
    out_shape = jax.ShapeDtypeStruct((1, seq_len, d_model), pe.dtype)
    return pl.pallas_call(
        _copy_body,
        grid=grid,
        in_specs=[pl.BlockSpec((1, _BLOCK_ROWS, d_model), lambda i: (0, i, 0))],
        out_specs=pl.BlockSpec((1, _BLOCK_ROWS, d_model), lambda i: (0, i, 0)),
        out_shape=out_shape,
    )(pe)
